# baseline (speedup 1.0000x reference)
import functools

import jax
import jax.numpy as jnp
from jax import lax
from jax.experimental import pallas as pl
from jax.experimental.pallas import tpu as pltpu
from jax.experimental.pallas import tpu_sc as plsc

N_NODES = 10000
D_FEAT = 128
NUM_GRAPHS = 64
NC = 2
NS = 16
SUB = 80
NSUB = N_NODES // SUB
ROUNDS = -(-NSUB // (NC * NS))


def _sc_pool_body(x_hbm, batch_hbm, zacc_hbm,
                  sums_hbm,
                  xbuf, idxbuf, lsems, isems,
                  acc):
    cid = lax.axis_index("c")
    sid = lax.axis_index("s")
    wid = cid * NS + sid
    rows = NUM_GRAPHS // NS

    pltpu.sync_copy(
        zacc_hbm.at[pl.ds(sid * rows, rows)], acc.at[pl.ds(sid * rows, rows)]
    )

    pltpu.make_async_copy(batch_hbm.at[wid], idxbuf, isems.at[0]).start()
    for i in range(ROUNDS):
        sc = wid * ROUNDS + i

        @pl.when(sc < NSUB)
        def _():
            pltpu.make_async_copy(
                x_hbm.at[pl.ds(sc * SUB, SUB)], xbuf.at[i], lsems.at[i]
            ).start()

    plsc.subcore_barrier()
    pltpu.make_async_copy(batch_hbm.at[wid], idxbuf, isems.at[0]).wait()

    for i in range(ROUNDS):
        sc = wid * ROUNDS + i

        @pl.when(sc < NSUB)
        def _():
            pltpu.make_async_copy(
                x_hbm.at[pl.ds(sc * SUB, SUB)], xbuf.at[i], lsems.at[i]
            ).wait()
            pltpu.sync_copy(xbuf.at[i], acc.at[idxbuf.at[i]], add=True)

    plsc.subcore_barrier()

    pltpu.sync_copy(
        acc.at[pl.ds(sid * rows, rows)],
        sums_hbm.at[cid].at[pl.ds(sid * rows, rows)],
    )


def _sc_pool(x, batch):
    zacc = jnp.zeros((NUM_GRAPHS, D_FEAT), jnp.float32)
    fn = functools.partial(
        pl.kernel,
        mesh=plsc.VectorSubcoreMesh(core_axis_name="c", subcore_axis_name="s"),
        out_type=jax.ShapeDtypeStruct((NC, NUM_GRAPHS, D_FEAT), jnp.float32),
        scratch_types=[
            pltpu.VMEM((ROUNDS, SUB, D_FEAT), jnp.float32),
            pltpu.VMEM((ROUNDS, SUB), jnp.int32),
            pltpu.SemaphoreType.DMA((ROUNDS,)),
            pltpu.SemaphoreType.DMA((1,)),
            pltpu.VMEM_SHARED((NUM_GRAPHS, D_FEAT), jnp.float32),
        ],
    )(_sc_pool_body)
    nw = NC * NS
    pad = nw * ROUNDS * SUB - batch.shape[0]
    batchp = jnp.concatenate([batch, jnp.zeros((pad,), jnp.int32)])
    return fn(x, batchp.reshape(nw, ROUNDS, SUB), zacc)


def _tc_counts_body(batch_ref, inv_ref):
    batch = batch_ref[0, :]
    gids = jax.lax.broadcasted_iota(jnp.int32, (NUM_GRAPHS, batch.shape[0]), 0)
    onehot = (batch[None, :] == gids).astype(jnp.float32)
    counts = jnp.sum(onehot, axis=1, keepdims=True)
    inv_ref[...] = 1.0 / jnp.maximum(counts, 1.0)


def _tc_mlp_body(sums_ref, inv_ref, W1_ref, b1_ref, W2_ref, b2_ref, out_ref):
    s = sums_ref[0] + sums_ref[1]
    pooled = s * inv_ref[...]
    h = jnp.maximum(
        jnp.dot(pooled, W1_ref[...], preferred_element_type=jnp.float32)
        + b1_ref[...],
        0.0,
    )
    out_ref[...] = (
        jnp.dot(h, W2_ref[...], preferred_element_type=jnp.float32) + b2_ref[...]
    )


def kernel(x, edge_index, batch, W1, b1, W2, b2):
    del edge_index
    sums2 = _sc_pool(x, batch)
    inv = pl.pallas_call(
        _tc_counts_body,
        out_shape=jax.ShapeDtypeStruct((NUM_GRAPHS, 1), jnp.float32),
    )(batch.reshape(1, -1))
    out = pl.pallas_call(
        _tc_mlp_body,
        out_shape=jax.ShapeDtypeStruct((NUM_GRAPHS, W2.shape[1]), jnp.float32),
    )(sums2, inv, W1, b1.reshape(1, -1), W2, b2.reshape(1, -1))
    return out
